# baseline (speedup 1.0000x reference)
import functools

import jax
import jax.numpy as jnp
from jax.experimental import pallas as pl
from jax.experimental.pallas import tpu as pltpu

_LOG2 = 16
_TILE = 1024


def _prep_body(embe_ref, embo_ref, wi_ref,
               ene_ref, eno_ref, c_ref, edb_ref):
    embe = embe_ref[...]
    embo = embo_ref[...]
    ene = embe / (jnp.sqrt(jnp.sum(embe * embe, axis=1, keepdims=True)) + 1e-8)
    eno = embo / (jnp.sqrt(jnp.sum(embo * embo, axis=1, keepdims=True)) + 1e-8)
    ene_ref[...] = ene.astype(jnp.bfloat16)
    eno_ref[...] = eno.astype(jnp.bfloat16)
    c_ref[...] = (jnp.sum(eno * eno, axis=1) -
                  jnp.sum(ene * ene, axis=1))[None, :]
    wi = wi_ref[...].astype(jnp.bfloat16)
    e2e = jax.lax.dot_general(
        ene.astype(jnp.bfloat16), wi, (((1,), (1,)), ((), ())),
        preferred_element_type=jnp.float32)
    e2o = jax.lax.dot_general(
        eno.astype(jnp.bfloat16), wi, (((1,), (1,)), ((), ())),
        preferred_element_type=jnp.float32)
    base = jnp.sum(e2o, axis=0, keepdims=True)
    edb_ref[...] = jnp.concatenate([e2e - e2o, base], axis=0).astype(
        jnp.bfloat16)


def _main_body(h_ref, m_ref, wp_ref, bp_ref, ene_ref, eno_ref, c_ref, edb_ref,
               bi_ref, out_ref, code_ref):
    h = h_ref[...]
    hp = jax.lax.dot_general(
        h.astype(jnp.bfloat16), wp_ref[...], (((1,), (1,)), ((), ())),
        preferred_element_type=jnp.float32) + bp_ref[...]
    hp = hp / (jnp.sqrt(jnp.sum(hp * hp, axis=1, keepdims=True)) + 1e-8)

    hpb = hp.astype(jnp.bfloat16)
    ce = jax.lax.dot_general(
        hpb, ene_ref[...], (((1,), (1,)), ((), ())),
        preferred_element_type=jnp.float32)
    co = jax.lax.dot_general(
        hpb, eno_ref[...], (((1,), (1,)), ((), ())),
        preferred_element_type=jnp.float32)
    dj = c_ref[...] - 2.0 * (co - ce)

    m = m_ref[...]
    sel = m == 1.0
    att = jnp.where(sel, jax.nn.sigmoid(dj), 0.0)
    mcol = jnp.where(sel, 1.0, 0.0)
    acat = jnp.concatenate([att, mcol], axis=1)
    out_ref[...] = jax.lax.dot_general(
        acat.astype(jnp.bfloat16), edb_ref[...], (((1,), (0,)), ((), ())),
        preferred_element_type=jnp.float32) + bi_ref[...]

    bits = (dj < 0.0).astype(jnp.int32)
    cidx = jax.lax.broadcasted_iota(jnp.int32, (1, _LOG2), 1)
    vq = jnp.sum(bits << cidx, axis=1, keepdims=True)
    code_ref[...] = jnp.where(sel, vq, 0)


@functools.partial(jax.jit, donate_argnums=())
def kernel(h, attn_mask, W_proj, b_proj, W_inv, b_inv, emb):
    Bb, Ss, D = h.shape
    V, _ = W_proj.shape
    N = Bb * Ss
    P = _LOG2
    hf = h.reshape(N, D)
    mf = attn_mask.reshape(N, 1)
    emb_e = emb[0::2]
    emb_o = emb[1::2]

    ene, eno, cvec, edb = pl.pallas_call(
        _prep_body,
        out_shape=[
            jax.ShapeDtypeStruct((P, V), jnp.bfloat16),
            jax.ShapeDtypeStruct((P, V), jnp.bfloat16),
            jax.ShapeDtypeStruct((1, P), jnp.float32),
            jax.ShapeDtypeStruct((P + 1, D), jnp.bfloat16),
        ],
    )(emb_e, emb_o, W_inv)

    grid = (N // _TILE,)
    out, codef = pl.pallas_call(
        _main_body,
        grid=grid,
        in_specs=[
            pl.BlockSpec((_TILE, D), lambda i: (i, 0)),
            pl.BlockSpec((_TILE, 1), lambda i: (i, 0)),
            pl.BlockSpec((V, D), lambda i: (0, 0)),
            pl.BlockSpec((1, V), lambda i: (0, 0)),
            pl.BlockSpec((P, V), lambda i: (0, 0)),
            pl.BlockSpec((P, V), lambda i: (0, 0)),
            pl.BlockSpec((1, P), lambda i: (0, 0)),
            pl.BlockSpec((P + 1, D), lambda i: (0, 0)),
            pl.BlockSpec((1, D), lambda i: (0, 0)),
        ],
        out_specs=[
            pl.BlockSpec((_TILE, D), lambda i: (i, 0)),
            pl.BlockSpec((_TILE, 1), lambda i: (i, 0)),
        ],
        out_shape=[
            jax.ShapeDtypeStruct((N, D), jnp.float32),
            jax.ShapeDtypeStruct((N, 1), jnp.int32),
        ],
        compiler_params=pltpu.CompilerParams(
            dimension_semantics=("parallel",)),
    )(hf, mf, W_proj.astype(jnp.bfloat16), b_proj.reshape(1, V),
      ene, eno, cvec, edb, b_inv.reshape(1, D))

    quantized = out.reshape(Bb, Ss, D)
    vq_code = codef.reshape(Bb, Ss)
    return (quantized, vq_code, jnp.float32(0.0))


# speedup vs baseline: 1.0162x; 1.0162x over previous
import functools

import jax
import jax.numpy as jnp
from jax.experimental import pallas as pl
from jax.experimental.pallas import tpu as pltpu

_LOG2 = 16
_TILE = 1024


def _prep_body(embe_ref, embo_ref, wi_ref, wp_ref, bi_ref,
               ene_ref, eno_ref, c_ref, edb_ref, wpb_ref, w2_ref):
    embe = embe_ref[...]
    embo = embo_ref[...]
    ene = embe / (jnp.sqrt(jnp.sum(embe * embe, axis=1, keepdims=True)) + 1e-8)
    eno = embo / (jnp.sqrt(jnp.sum(embo * embo, axis=1, keepdims=True)) + 1e-8)
    ene_ref[...] = ene.astype(jnp.bfloat16)
    eno_ref[...] = eno.astype(jnp.bfloat16)
    c_ref[...] = (jnp.sum(eno * eno, axis=1) -
                  jnp.sum(ene * ene, axis=1))[None, :]
    wi = wi_ref[...].astype(jnp.bfloat16)
    e2e = jax.lax.dot_general(
        ene.astype(jnp.bfloat16), wi, (((1,), (1,)), ((), ())),
        preferred_element_type=jnp.float32)
    e2o = jax.lax.dot_general(
        eno.astype(jnp.bfloat16), wi, (((1,), (1,)), ((), ())),
        preferred_element_type=jnp.float32)
    base = jnp.sum(e2o, axis=0, keepdims=True)
    edb_ref[...] = jnp.concatenate(
        [e2e - e2o, base, bi_ref[...]], axis=0).astype(jnp.bfloat16)
    wpb_ref[...] = wp_ref[...].astype(jnp.bfloat16)
    ridx = jax.lax.broadcasted_iota(jnp.int32, (_LOG2, 1), 0)
    w2_ref[...] = jnp.exp2(ridx.astype(jnp.float32)).astype(jnp.bfloat16)


def _main_body(h_ref, m_ref, wp_ref, bp_ref, ene_ref, eno_ref, c_ref, edb_ref,
               w2_ref, out_ref, code_ref):
    h = h_ref[...]
    hp = jax.lax.dot_general(
        h.astype(jnp.bfloat16), wp_ref[...], (((1,), (1,)), ((), ())),
        preferred_element_type=jnp.float32) + bp_ref[...]
    hp = hp / (jnp.sqrt(jnp.sum(hp * hp, axis=1, keepdims=True)) + 1e-8)

    hpb = hp.astype(jnp.bfloat16)
    ce = jax.lax.dot_general(
        hpb, ene_ref[...], (((1,), (1,)), ((), ())),
        preferred_element_type=jnp.float32)
    co = jax.lax.dot_general(
        hpb, eno_ref[...], (((1,), (1,)), ((), ())),
        preferred_element_type=jnp.float32)
    dj = c_ref[...] - 2.0 * (co - ce)

    m = m_ref[...]
    sel = m == 1.0
    att = jnp.where(sel, jax.nn.sigmoid(dj), 0.0)
    mcol = jnp.where(sel, 1.0, 0.0)
    ones = jnp.ones_like(mcol)
    acat = jnp.concatenate([att, mcol, ones], axis=1)
    out_ref[...] = jax.lax.dot_general(
        acat.astype(jnp.bfloat16), edb_ref[...], (((1,), (0,)), ((), ())),
        preferred_element_type=jnp.float32)

    bits = jnp.where(dj < 0.0, 1.0, 0.0).astype(jnp.bfloat16)
    vqf = jax.lax.dot_general(
        bits, w2_ref[...], (((1,), (0,)), ((), ())),
        preferred_element_type=jnp.float32)
    code_ref[...] = jnp.where(sel, vqf, 0.0).astype(jnp.int32)


@functools.partial(jax.jit, donate_argnums=())
def kernel(h, attn_mask, W_proj, b_proj, W_inv, b_inv, emb):
    Bb, Ss, D = h.shape
    V, _ = W_proj.shape
    N = Bb * Ss
    P = _LOG2
    hf = h.reshape(N, D)
    mf = attn_mask.reshape(N, 1)
    emb_e = emb[0::2]
    emb_o = emb[1::2]

    ene, eno, cvec, edb, wpb, w2 = pl.pallas_call(
        _prep_body,
        out_shape=[
            jax.ShapeDtypeStruct((P, V), jnp.bfloat16),
            jax.ShapeDtypeStruct((P, V), jnp.bfloat16),
            jax.ShapeDtypeStruct((1, P), jnp.float32),
            jax.ShapeDtypeStruct((P + 2, D), jnp.bfloat16),
            jax.ShapeDtypeStruct((V, D), jnp.bfloat16),
            jax.ShapeDtypeStruct((P, 1), jnp.bfloat16),
        ],
    )(emb_e, emb_o, W_inv, W_proj, b_inv.reshape(1, D))

    grid = (N // _TILE,)
    out, codef = pl.pallas_call(
        _main_body,
        grid=grid,
        in_specs=[
            pl.BlockSpec((_TILE, D), lambda i: (i, 0)),
            pl.BlockSpec((_TILE, 1), lambda i: (i, 0)),
            pl.BlockSpec((V, D), lambda i: (0, 0)),
            pl.BlockSpec((1, V), lambda i: (0, 0)),
            pl.BlockSpec((P, V), lambda i: (0, 0)),
            pl.BlockSpec((P, V), lambda i: (0, 0)),
            pl.BlockSpec((1, P), lambda i: (0, 0)),
            pl.BlockSpec((P + 2, D), lambda i: (0, 0)),
            pl.BlockSpec((P, 1), lambda i: (0, 0)),
        ],
        out_specs=[
            pl.BlockSpec((_TILE, D), lambda i: (i, 0)),
            pl.BlockSpec((_TILE, 1), lambda i: (i, 0)),
        ],
        out_shape=[
            jax.ShapeDtypeStruct((N, D), jnp.float32),
            jax.ShapeDtypeStruct((N, 1), jnp.int32),
        ],
        compiler_params=pltpu.CompilerParams(
            dimension_semantics=("parallel",)),
    )(hf, mf, wpb, b_proj.reshape(1, V), ene, eno, cvec, edb, w2)

    quantized = out.reshape(Bb, Ss, D)
    vq_code = codef.reshape(Bb, Ss)
    return (quantized, vq_code, jnp.float32(0.0))
